# Initial kernel scaffold; baseline (speedup 1.0000x reference)
import jax
import jax.numpy as jnp
from jax.experimental import pallas as pl


def kernel(x, edge_index, batch, W1, b1, g1, be1, W2, b2, g2, be2, W3, b3, g3, be3):
    raise NotImplementedError("write your pallas kernel here")


# speedup vs baseline: 11.9283x; 11.9283x over previous
import jax
import jax.numpy as jnp
from jax import lax
from jax.experimental import pallas as pl
from jax.experimental.pallas import tpu as pltpu
from jax.experimental.pallas import tpu_sc as plsc

N = 100000
B = 128
EPS = 1e-5
SLOPE = 0.01

NC = 2
NS = 16
LANES = 16

NPAD = 102400
STRIPE = NPAD // NS
ZCH = 640

GROUP = 512
SUB = 128
GSUB = GROUP // SUB

GP_SPMM = 196
EPAD = NS * GP_SPMM * GROUP
GP_DEG = EPAD // (NC * NS * GROUP)

RB = 2000
NRB = N // RB


_MESH = plsc.VectorSubcoreMesh(core_axis_name="c", subcore_axis_name="s")
_SC_PARAMS = pltpu.CompilerParams(use_tc_tiling_on_sc=False)


def _deg_body(col2, degp, acc, zb, colg, ones_v):
    c = lax.axis_index("c")
    s = lax.axis_index("s")

    @pl.loop(0, STRIPE // LANES)
    def _(i):
        zb[pl.ds(i * LANES, LANES)] = jnp.zeros((LANES,), jnp.float32)

    @pl.loop(0, SUB // LANES)
    def _(i):
        ones_v[pl.ds(i * LANES, LANES)] = jnp.ones((LANES,), jnp.float32)

    pltpu.sync_copy(zb, acc.at[pl.ds(pl.multiple_of(s * STRIPE, STRIPE), STRIPE)])
    plsc.subcore_barrier()

    w = s * NC + c
    base = w * GP_DEG * GSUB

    @pl.loop(0, GP_DEG)
    def _(g):
        pltpu.sync_copy(
            col2.at[pl.ds(pl.multiple_of(base + g * GSUB, GSUB), GSUB)], colg)
        for u in range(GSUB):
            pltpu.sync_copy(ones_v, acc.at[colg.at[u]], add=True)

    plsc.subcore_barrier()
    pltpu.sync_copy(
        acc.at[pl.ds(pl.multiple_of(s * STRIPE, STRIPE), STRIPE)],
        degp.at[pl.ds(pl.multiple_of(c * NPAD + s * STRIPE, STRIPE), STRIPE)])


_deg_call = pl.kernel(
    _deg_body,
    out_type=jax.ShapeDtypeStruct((NC * NPAD,), jnp.float32),
    mesh=_MESH,
    compiler_params=_SC_PARAMS,
    scratch_types=[
        pltpu.VMEM_SHARED((NPAD,), jnp.float32),
        pltpu.VMEM((STRIPE,), jnp.float32),
        pltpu.VMEM((GSUB, SUB), jnp.int32),
        pltpu.VMEM((SUB,), jnp.float32),
    ],
)


def _make_spmm(nblk):
    bpc = nblk // NC

    def body(yf, row_h, col2, z, acc, zb, rowg, rowa, colg, gbuf, sem):
        c = lax.axis_index("c")
        s = lax.axis_index("s")

        @pl.loop(0, ZCH)
        def _(i):
            zb[i] = jnp.zeros((LANES,), jnp.float32)

        for bi in range(bpc):
            blk = bi * NC + c

            @pl.loop(0, STRIPE // ZCH)
            def _(k):
                pltpu.sync_copy(
                    zb,
                    acc.at[pl.ds(pl.multiple_of(s * STRIPE + k * ZCH, ZCH), ZCH)])

            plsc.subcore_barrier()

            ebase = s * GP_SPMM * GROUP

            @pl.loop(0, GP_SPMM)
            def _(g):
                gb = pl.multiple_of(ebase + g * GROUP, GROUP)
                gb128 = pl.multiple_of(s * GP_SPMM * GSUB + g * GSUB, GSUB)
                pltpu.sync_copy(row_h.at[pl.ds(gb, GROUP)], rowg)
                pltpu.sync_copy(col2.at[pl.ds(gb128, GSUB)], colg)

                @pl.loop(0, GROUP // LANES)
                def _(j):
                    rowa[pl.ds(j * LANES, LANES)] = (
                        rowg[pl.ds(j * LANES, LANES)] * nblk + blk)

                cps = []
                for u in range(GSUB):
                    cps.append(pltpu.async_copy(
                        yf.at[rowa.at[pl.ds(u * SUB, SUB)]],
                        gbuf.at[pl.ds(u * SUB, SUB)], sem))
                for cp in cps:
                    cp.wait()
                for u in range(GSUB):
                    pltpu.sync_copy(gbuf.at[pl.ds(u * SUB, SUB)],
                                    acc.at[colg.at[u]], add=True)

            plsc.subcore_barrier()

            @pl.loop(0, STRIPE // ZCH)
            def _(k):
                r = pl.ds(pl.multiple_of(s * STRIPE + k * ZCH, ZCH), ZCH)
                pltpu.sync_copy(acc.at[r], z.at[blk, r])

            plsc.subcore_barrier()

    return pl.kernel(
        body,
        out_type=jax.ShapeDtypeStruct((nblk, NPAD, LANES), jnp.float32),
        mesh=_MESH,
        compiler_params=_SC_PARAMS,
        scratch_types=[
            pltpu.VMEM_SHARED((NPAD, LANES), jnp.float32),
            pltpu.VMEM((ZCH, LANES), jnp.float32),
            pltpu.VMEM((GROUP,), jnp.int32),
            pltpu.VMEM((GROUP,), jnp.int32),
            pltpu.VMEM((GSUB, SUB), jnp.int32),
            pltpu.VMEM((GROUP, LANES), jnp.float32),
            pltpu.SemaphoreType.DMA,
        ],
    )


_spmm64 = _make_spmm(4)
_spmm32 = _make_spmm(2)


def _t1_body(x_ref, w_ref, d0_ref, d1_ref, y_ref):
    dinv = lax.rsqrt(d0_ref[...] + d1_ref[...] + 1.0)
    y_ref[...] = jnp.dot(x_ref[...], w_ref[...],
                         preferred_element_type=jnp.float32) * dinv


def _t1_call(din, dout):
    return pl.pallas_call(
        _t1_body,
        grid=(NRB,),
        in_specs=[
            pl.BlockSpec((RB, din), lambda i: (i, 0)),
            pl.BlockSpec((din, dout), lambda i: (0, 0)),
            pl.BlockSpec((RB, 1), lambda i: (i, 0)),
            pl.BlockSpec((RB, 1), lambda i: (i, 0)),
        ],
        out_specs=pl.BlockSpec((RB, dout), lambda i: (i, 0)),
        out_shape=jax.ShapeDtypeStruct((N, dout), jnp.float32),
    )


def _make_t2_body(nblk):
    def _t2_body(z_ref, y_ref, d0_ref, d1_ref, b_ref, o_ref, st_ref):
        i = pl.program_id(0)
        dinv = lax.rsqrt(d0_ref[...] + d1_ref[...] + 1.0)
        zcat = jnp.concatenate([z_ref[j] for j in range(nblk)], axis=1)
        o = (zcat + y_ref[...]) * dinv + b_ref[...]
        o_ref[...] = o

        @pl.when(i == 0)
        def _():
            st_ref[...] = jnp.zeros_like(st_ref)

        st_ref[0:1, :] += jnp.sum(o, axis=0, keepdims=True)
        st_ref[1:2, :] += jnp.sum(o * o, axis=0, keepdims=True)

    return _t2_body


def _t2_call(d):
    nblk = d // LANES
    return pl.pallas_call(
        _make_t2_body(nblk),
        grid=(NRB,),
        in_specs=[
            pl.BlockSpec((nblk, RB, LANES), lambda i: (0, i, 0)),
            pl.BlockSpec((RB, d), lambda i: (i, 0)),
            pl.BlockSpec((RB, 1), lambda i: (i, 0)),
            pl.BlockSpec((RB, 1), lambda i: (i, 0)),
            pl.BlockSpec((1, d), lambda i: (0, 0)),
        ],
        out_specs=[
            pl.BlockSpec((RB, d), lambda i: (i, 0)),
            pl.BlockSpec((8, d), lambda i: (0, 0)),
        ],
        out_shape=[
            jax.ShapeDtypeStruct((N, d), jnp.float32),
            jax.ShapeDtypeStruct((8, d), jnp.float32),
        ],
    )


def _bn_act(o_ref, st_ref, g_ref, be_ref):
    mean = st_ref[0:1, :] * (1.0 / N)
    var = st_ref[1:2, :] * (1.0 / N) - mean * mean
    xn = (o_ref[...] - mean) * lax.rsqrt(var + EPS) * g_ref[...] + be_ref[...]
    return jnp.where(xn >= 0, xn, SLOPE * xn)


def _t3_body(o_ref, st_ref, g_ref, be_ref, w_ref, d0_ref, d1_ref, y_ref):
    h = _bn_act(o_ref, st_ref, g_ref, be_ref)
    dinv = lax.rsqrt(d0_ref[...] + d1_ref[...] + 1.0)
    y_ref[...] = jnp.dot(h, w_ref[...],
                         preferred_element_type=jnp.float32) * dinv


def _t3_call(din, dout):
    return pl.pallas_call(
        _t3_body,
        grid=(NRB,),
        in_specs=[
            pl.BlockSpec((RB, din), lambda i: (i, 0)),
            pl.BlockSpec((8, din), lambda i: (0, 0)),
            pl.BlockSpec((1, din), lambda i: (0, 0)),
            pl.BlockSpec((1, din), lambda i: (0, 0)),
            pl.BlockSpec((din, dout), lambda i: (0, 0)),
            pl.BlockSpec((RB, 1), lambda i: (i, 0)),
            pl.BlockSpec((RB, 1), lambda i: (i, 0)),
        ],
        out_specs=pl.BlockSpec((RB, dout), lambda i: (i, 0)),
        out_shape=jax.ShapeDtypeStruct((N, dout), jnp.float32),
    )


def _t4_body(o_ref, st_ref, g_ref, be_ref, bt_ref, out_ref, acc, cnt):
    i = pl.program_id(0)

    @pl.when(i == 0)
    def _():
        acc[...] = jnp.zeros_like(acc)
        cnt[...] = jnp.zeros_like(cnt)

    h = _bn_act(o_ref, st_ref, g_ref, be_ref)
    onehot = (bt_ref[...] == lax.broadcasted_iota(jnp.int32, (RB, B), 1)
              ).astype(jnp.float32)
    acc[...] += lax.dot_general(onehot, h, (((0,), (0,)), ((), ())),
                                preferred_element_type=jnp.float32)
    cnt[...] += lax.dot_general(onehot, jnp.ones((RB, 1), jnp.float32),
                                (((0,), (0,)), ((), ())),
                                preferred_element_type=jnp.float32)

    @pl.when(i == NRB - 1)
    def _():
        out_ref[...] = acc[...] / jnp.maximum(cnt[...], 1.0)


def _t4_call(d):
    return pl.pallas_call(
        _t4_body,
        grid=(NRB,),
        in_specs=[
            pl.BlockSpec((RB, d), lambda i: (i, 0)),
            pl.BlockSpec((8, d), lambda i: (0, 0)),
            pl.BlockSpec((1, d), lambda i: (0, 0)),
            pl.BlockSpec((1, d), lambda i: (0, 0)),
            pl.BlockSpec((RB, 1), lambda i: (i, 0)),
        ],
        out_specs=pl.BlockSpec((B, d), lambda i: (0, 0)),
        out_shape=jax.ShapeDtypeStruct((B, d), jnp.float32),
        scratch_shapes=[
            pltpu.VMEM((B, d), jnp.float32),
            pltpu.VMEM((B, 1), jnp.float32),
        ],
    )


def kernel(x, edge_index, batch,
           W1, b1, g1, be1, W2, b2, g2, be2, W3, b3, g3, be3):
    row = edge_index[0].astype(jnp.int32)
    col = edge_index[1].astype(jnp.int32)
    e = row.shape[0]
    row_p = jnp.concatenate([row, jnp.zeros((EPAD - e,), jnp.int32)])
    col_p = jnp.concatenate([col, jnp.full((EPAD - e,), NPAD - 1, jnp.int32)])
    col2 = col_p.reshape(EPAD // SUB, SUB)
    batch_c = batch.astype(jnp.int32).reshape(N, 1)

    x8 = jnp.pad(x, ((0, 0), (0, 8 - x.shape[1])))
    W18 = jnp.pad(W1, ((0, 8 - W1.shape[0]), (0, 0)))

    degp = _deg_call(col2).reshape(NC, NPAD, 1)
    d0 = degp[0]
    d1 = degp[1]

    hid = W1.shape[1]
    emb = W3.shape[1]

    y1 = _t1_call(8, hid)(x8, W18, d0, d1)
    z1 = _spmm64(y1.reshape(N * 4, LANES), row_p, col2)
    o1, st1 = _t2_call(hid)(z1, y1, d0, d1, b1.reshape(1, hid))
    y2 = _t3_call(hid, hid)(o1, st1, g1.reshape(1, hid), be1.reshape(1, hid),
                            W2, d0, d1)
    z2 = _spmm64(y2.reshape(N * 4, LANES), row_p, col2)
    o2, st2 = _t2_call(hid)(z2, y2, d0, d1, b2.reshape(1, hid))
    y3 = _t3_call(hid, emb)(o2, st2, g2.reshape(1, hid), be2.reshape(1, hid),
                            W3, d0, d1)
    z3 = _spmm32(y3.reshape(N * 2, LANES), row_p, col2)
    o3, st3 = _t2_call(emb)(z3, y3, d0, d1, b3.reshape(1, emb))
    out = _t4_call(emb)(o3, st3, g3.reshape(1, emb), be3.reshape(1, emb),
                        batch_c)
    return out


# speedup vs baseline: 17.2504x; 1.4462x over previous
import jax
import jax.numpy as jnp
from jax import lax
from jax.experimental import pallas as pl
from jax.experimental.pallas import tpu as pltpu
from jax.experimental.pallas import tpu_sc as plsc

N = 100000
B = 128
EPS = 1e-5
SLOPE = 0.01

NC = 2
NS = 16
LANES = 16

NPAD = 100352
STRIPE = NPAD // NS
ZCH = 392

GROUP = 512
SUB = 128
GSUB = GROUP // SUB

GP_SPMM = 196
EPAD = NS * GP_SPMM * GROUP
GP_DEG = EPAD // (NC * NS * GROUP)

RB = 2000
NRB = N // RB


_MESH = plsc.VectorSubcoreMesh(core_axis_name="c", subcore_axis_name="s")
_SC_PARAMS = pltpu.CompilerParams(use_tc_tiling_on_sc=False)


def _deg_body(col2, degp, acc, zb, colg, ones_v, isem, ssem):
    c = lax.axis_index("c")
    s = lax.axis_index("s")

    @pl.loop(0, STRIPE // LANES)
    def _(i):
        zb[pl.ds(i * LANES, LANES)] = jnp.zeros((LANES,), jnp.float32)

    @pl.loop(0, SUB // LANES)
    def _(i):
        ones_v[pl.ds(i * LANES, LANES)] = jnp.ones((LANES,), jnp.float32)

    pltpu.sync_copy(zb, acc.at[pl.ds(pl.multiple_of(s * STRIPE, STRIPE), STRIPE)])
    plsc.subcore_barrier()

    w = s * NC + c
    base = w * GP_DEG * GSUB
    psub = 2 * GSUB

    @pl.loop(0, GP_DEG // 2)
    def _(g):
        pltpu.sync_copy(
            col2.at[pl.ds(pl.multiple_of(base + g * psub, psub), psub)], colg)
        scps = [pltpu.async_copy(ones_v, acc.at[colg.at[u]], ssem, add=True)
                for u in range(psub)]
        for cp in scps:
            cp.wait()

    plsc.subcore_barrier()
    pltpu.sync_copy(
        acc.at[pl.ds(pl.multiple_of(s * STRIPE, STRIPE), STRIPE)],
        degp.at[pl.ds(pl.multiple_of(c * NPAD + s * STRIPE, STRIPE), STRIPE)])


_deg_call = pl.kernel(
    _deg_body,
    out_type=jax.ShapeDtypeStruct((NC * NPAD,), jnp.float32),
    mesh=_MESH,
    compiler_params=_SC_PARAMS,
    scratch_types=[
        pltpu.VMEM_SHARED((NPAD,), jnp.float32),
        pltpu.VMEM((STRIPE,), jnp.float32),
        pltpu.VMEM((2 * GSUB, SUB), jnp.int32),
        pltpu.VMEM((SUB,), jnp.float32),
        pltpu.SemaphoreType.DMA,
        pltpu.SemaphoreType.DMA,
    ],
)


def _make_spmm(nblk):
    bpc = nblk // NC
    pair = 2 * GROUP
    psub = 2 * GSUB

    def body(yf, row_h, col2, z, acc, zb, rowp, colp, gbuf0, gbuf1,
             isem, gsem0, gsem1, ssem0, ssem1, wsem):
        c = lax.axis_index("c")
        s = lax.axis_index("s")

        @pl.loop(0, ZCH)
        def _(i):
            zb[i] = jnp.zeros((LANES,), jnp.float32)

        for bi in range(bpc):
            blk = bi * NC + c

            zcps = [
                pltpu.async_copy(
                    zb,
                    acc.at[pl.ds(pl.multiple_of(s * STRIPE + k * ZCH, ZCH),
                                 ZCH)],
                    wsem)
                for k in range(STRIPE // ZCH)
            ]
            for cp in zcps:
                cp.wait()
            plsc.subcore_barrier()

            @pl.loop(0, GP_SPMM // 2)
            def _(i):
                eb = pl.multiple_of(s * GP_SPMM * GROUP + i * pair, pair)
                eb128 = pl.multiple_of(s * GP_SPMM * GSUB + i * psub, psub)
                icps = [pltpu.async_copy(row_h.at[pl.ds(eb, pair)], rowp,
                                         isem),
                        pltpu.async_copy(col2.at[pl.ds(eb128, psub)], colp,
                                         isem)]
                for cp in icps:
                    cp.wait()

                @pl.loop(0, pair // LANES)
                def _(j):
                    sl = pl.ds(j * LANES, LANES)
                    rowp[sl] = rowp[sl] * nblk + blk

                gcps0 = [pltpu.async_copy(
                    yf.at[rowp.at[pl.ds(u * SUB, SUB)]],
                    gbuf0.at[pl.ds(u * SUB, SUB)], gsem0)
                    for u in range(GSUB)]
                gcps1 = [pltpu.async_copy(
                    yf.at[rowp.at[pl.ds(GROUP + u * SUB, SUB)]],
                    gbuf1.at[pl.ds(u * SUB, SUB)], gsem1)
                    for u in range(GSUB)]
                for cp in gcps0:
                    cp.wait()
                scps0 = [pltpu.async_copy(
                    gbuf0.at[pl.ds(u * SUB, SUB)],
                    acc.at[colp.at[u]], ssem0, add=True)
                    for u in range(GSUB)]
                for cp in gcps1:
                    cp.wait()
                scps1 = [pltpu.async_copy(
                    gbuf1.at[pl.ds(u * SUB, SUB)],
                    acc.at[colp.at[GSUB + u]], ssem1, add=True)
                    for u in range(GSUB)]
                for cp in scps0:
                    cp.wait()
                for cp in scps1:
                    cp.wait()

            plsc.subcore_barrier()

            r = pl.ds(pl.multiple_of(s * STRIPE, STRIPE), STRIPE)
            pltpu.sync_copy(acc.at[r], z.at[blk, r])
            plsc.subcore_barrier()

    return pl.kernel(
        body,
        out_type=jax.ShapeDtypeStruct((nblk, NPAD, LANES), jnp.float32),
        mesh=_MESH,
        compiler_params=_SC_PARAMS,
        scratch_types=[
            pltpu.VMEM_SHARED((NPAD, LANES), jnp.float32),
            pltpu.VMEM((ZCH, LANES), jnp.float32),
            pltpu.VMEM((pair,), jnp.int32),
            pltpu.VMEM((psub, SUB), jnp.int32),
            pltpu.VMEM((GROUP, LANES), jnp.float32),
            pltpu.VMEM((GROUP, LANES), jnp.float32),
            pltpu.SemaphoreType.DMA,
            pltpu.SemaphoreType.DMA,
            pltpu.SemaphoreType.DMA,
            pltpu.SemaphoreType.DMA,
            pltpu.SemaphoreType.DMA,
            pltpu.SemaphoreType.DMA,
        ],
    )


_spmm64 = _make_spmm(4)
_spmm32 = _make_spmm(2)


def _t1_body(x_ref, w_ref, d0_ref, d1_ref, y_ref):
    dinv = lax.rsqrt(d0_ref[...] + d1_ref[...] + 1.0)
    y_ref[...] = jnp.dot(x_ref[...], w_ref[...],
                         preferred_element_type=jnp.float32) * dinv


def _t1_call(din, dout):
    return pl.pallas_call(
        _t1_body,
        grid=(NRB,),
        in_specs=[
            pl.BlockSpec((RB, din), lambda i: (i, 0)),
            pl.BlockSpec((din, dout), lambda i: (0, 0)),
            pl.BlockSpec((RB, 1), lambda i: (i, 0)),
            pl.BlockSpec((RB, 1), lambda i: (i, 0)),
        ],
        out_specs=pl.BlockSpec((RB, dout), lambda i: (i, 0)),
        out_shape=jax.ShapeDtypeStruct((N, dout), jnp.float32),
    )


def _make_t2_body(nblk):
    def _t2_body(z_ref, y_ref, d0_ref, d1_ref, b_ref, o_ref, st_ref):
        i = pl.program_id(0)
        dinv = lax.rsqrt(d0_ref[...] + d1_ref[...] + 1.0)
        zcat = jnp.concatenate([z_ref[j] for j in range(nblk)], axis=1)
        o = (zcat + y_ref[...]) * dinv + b_ref[...]
        o_ref[...] = o

        @pl.when(i == 0)
        def _():
            st_ref[...] = jnp.zeros_like(st_ref)

        st_ref[0:1, :] += jnp.sum(o, axis=0, keepdims=True)
        st_ref[1:2, :] += jnp.sum(o * o, axis=0, keepdims=True)

    return _t2_body


def _t2_call(d):
    nblk = d // LANES
    return pl.pallas_call(
        _make_t2_body(nblk),
        grid=(NRB,),
        in_specs=[
            pl.BlockSpec((nblk, RB, LANES), lambda i: (0, i, 0)),
            pl.BlockSpec((RB, d), lambda i: (i, 0)),
            pl.BlockSpec((RB, 1), lambda i: (i, 0)),
            pl.BlockSpec((RB, 1), lambda i: (i, 0)),
            pl.BlockSpec((1, d), lambda i: (0, 0)),
        ],
        out_specs=[
            pl.BlockSpec((RB, d), lambda i: (i, 0)),
            pl.BlockSpec((8, d), lambda i: (0, 0)),
        ],
        out_shape=[
            jax.ShapeDtypeStruct((N, d), jnp.float32),
            jax.ShapeDtypeStruct((8, d), jnp.float32),
        ],
    )


def _bn_act(o_ref, st_ref, g_ref, be_ref):
    mean = st_ref[0:1, :] * (1.0 / N)
    var = st_ref[1:2, :] * (1.0 / N) - mean * mean
    xn = (o_ref[...] - mean) * lax.rsqrt(var + EPS) * g_ref[...] + be_ref[...]
    return jnp.where(xn >= 0, xn, SLOPE * xn)


def _t3_body(o_ref, st_ref, g_ref, be_ref, w_ref, d0_ref, d1_ref, y_ref):
    h = _bn_act(o_ref, st_ref, g_ref, be_ref)
    dinv = lax.rsqrt(d0_ref[...] + d1_ref[...] + 1.0)
    y_ref[...] = jnp.dot(h, w_ref[...],
                         preferred_element_type=jnp.float32) * dinv


def _t3_call(din, dout):
    return pl.pallas_call(
        _t3_body,
        grid=(NRB,),
        in_specs=[
            pl.BlockSpec((RB, din), lambda i: (i, 0)),
            pl.BlockSpec((8, din), lambda i: (0, 0)),
            pl.BlockSpec((1, din), lambda i: (0, 0)),
            pl.BlockSpec((1, din), lambda i: (0, 0)),
            pl.BlockSpec((din, dout), lambda i: (0, 0)),
            pl.BlockSpec((RB, 1), lambda i: (i, 0)),
            pl.BlockSpec((RB, 1), lambda i: (i, 0)),
        ],
        out_specs=pl.BlockSpec((RB, dout), lambda i: (i, 0)),
        out_shape=jax.ShapeDtypeStruct((N, dout), jnp.float32),
    )


def _t4_body(o_ref, st_ref, g_ref, be_ref, bt_ref, out_ref, acc, cnt):
    i = pl.program_id(0)

    @pl.when(i == 0)
    def _():
        acc[...] = jnp.zeros_like(acc)
        cnt[...] = jnp.zeros_like(cnt)

    h = _bn_act(o_ref, st_ref, g_ref, be_ref)
    onehot = (bt_ref[...] == lax.broadcasted_iota(jnp.int32, (RB, B), 1)
              ).astype(jnp.float32)
    acc[...] += lax.dot_general(onehot, h, (((0,), (0,)), ((), ())),
                                preferred_element_type=jnp.float32)
    cnt[...] += lax.dot_general(onehot, jnp.ones((RB, 1), jnp.float32),
                                (((0,), (0,)), ((), ())),
                                preferred_element_type=jnp.float32)

    @pl.when(i == NRB - 1)
    def _():
        out_ref[...] = acc[...] / jnp.maximum(cnt[...], 1.0)


def _t4_call(d):
    return pl.pallas_call(
        _t4_body,
        grid=(NRB,),
        in_specs=[
            pl.BlockSpec((RB, d), lambda i: (i, 0)),
            pl.BlockSpec((8, d), lambda i: (0, 0)),
            pl.BlockSpec((1, d), lambda i: (0, 0)),
            pl.BlockSpec((1, d), lambda i: (0, 0)),
            pl.BlockSpec((RB, 1), lambda i: (i, 0)),
        ],
        out_specs=pl.BlockSpec((B, d), lambda i: (0, 0)),
        out_shape=jax.ShapeDtypeStruct((B, d), jnp.float32),
        scratch_shapes=[
            pltpu.VMEM((B, d), jnp.float32),
            pltpu.VMEM((B, 1), jnp.float32),
        ],
    )


def kernel(x, edge_index, batch,
           W1, b1, g1, be1, W2, b2, g2, be2, W3, b3, g3, be3):
    row = edge_index[0].astype(jnp.int32)
    col = edge_index[1].astype(jnp.int32)
    e = row.shape[0]
    row_p = jnp.concatenate([row, jnp.zeros((EPAD - e,), jnp.int32)])
    col_p = jnp.concatenate([col, jnp.full((EPAD - e,), NPAD - 1, jnp.int32)])
    col2 = col_p.reshape(EPAD // SUB, SUB)
    batch_c = batch.astype(jnp.int32).reshape(N, 1)

    x8 = jnp.pad(x, ((0, 0), (0, 8 - x.shape[1])))
    W18 = jnp.pad(W1, ((0, 8 - W1.shape[0]), (0, 0)))

    degp = _deg_call(col2).reshape(NC, NPAD, 1)
    d0 = degp[0]
    d1 = degp[1]

    hid = W1.shape[1]
    emb = W3.shape[1]

    y1 = _t1_call(8, hid)(x8, W18, d0, d1)
    z1 = _spmm64(y1.reshape(N * 4, LANES), row_p, col2)
    o1, st1 = _t2_call(hid)(z1, y1, d0, d1, b1.reshape(1, hid))
    y2 = _t3_call(hid, hid)(o1, st1, g1.reshape(1, hid), be1.reshape(1, hid),
                            W2, d0, d1)
    z2 = _spmm64(y2.reshape(N * 4, LANES), row_p, col2)
    o2, st2 = _t2_call(hid)(z2, y2, d0, d1, b2.reshape(1, hid))
    y3 = _t3_call(hid, emb)(o2, st2, g2.reshape(1, hid), be2.reshape(1, hid),
                            W3, d0, d1)
    z3 = _spmm32(y3.reshape(N * 2, LANES), row_p, col2)
    o3, st3 = _t2_call(emb)(z3, y3, d0, d1, b3.reshape(1, emb))
    out = _t4_call(emb)(o3, st3, g3.reshape(1, emb), be3.reshape(1, emb),
                        batch_c)
    return out
